# baseline (speedup 1.0000x reference)
import functools

import numpy as np
import jax
import jax.numpy as jnp
from jax import lax
from jax.experimental import pallas as pl
from jax.experimental.pallas import tpu as pltpu
from jax.experimental.pallas import tpu_sc as plsc

PI = np.pi
CUTOFF = 5.0

N, E, H, F, R = 10000, 160000, 512, 512, 64
NC, NS, L = 2, 16, 16
FH = F // NC
FQ = F // 4
NB = 2 * NS
BKT = 320
NPAD = NB * BKT
E_PER_TILE = E // NS
CAP = 1008
CH2 = 64
AGG_ROWS = BKT + 1


def _node_stage_body(x_ref, wnt_ref, b_ref, g_ref, beta_ref, wl1t_ref, h1_ref):
    xb = x_ref[...]
    h = jnp.dot(xb, wnt_ref[...], preferred_element_type=jnp.float32) + b_ref[...]
    mu = jnp.mean(h, axis=-1, keepdims=True)
    var = jnp.mean((h - mu) ** 2, axis=-1, keepdims=True)
    hn = (h - mu) * lax.rsqrt(var + 1e-5) * g_ref[...] + beta_ref[...]
    h1 = jnp.dot(hn, wl1t_ref[...], preferred_element_type=jnp.float32)
    w = lax.bitcast_convert_type(h1.astype(jnp.bfloat16),
                                 jnp.uint16).astype(jnp.int32)
    h1_ref[0] = w[:, 0:FQ] | (w[:, FQ:2 * FQ] << 16)
    h1_ref[1] = w[:, 2 * FQ:3 * FQ] | (w[:, 3 * FQ:4 * FQ] << 16)


def _node_stage(x, W_n2mT, b_n2m, ln_g, ln_b, W_lin1T):
    BN = 2000
    grid = N // BN
    return pl.pallas_call(
        _node_stage_body,
        grid=(grid,),
        in_specs=[
            pl.BlockSpec((BN, H), lambda i: (i, 0)),
            pl.BlockSpec((H, F), lambda i: (0, 0)),
            pl.BlockSpec((1, F), lambda i: (0, 0)),
            pl.BlockSpec((1, F), lambda i: (0, 0)),
            pl.BlockSpec((1, F), lambda i: (0, 0)),
            pl.BlockSpec((F, F), lambda i: (0, 0)),
        ],
        out_specs=pl.BlockSpec((NC, BN, FQ), lambda i: (0, i, 0)),
        out_shape=jax.ShapeDtypeStruct((NC, N, FQ), jnp.int32),
    )(x, W_n2mT, b_n2m.reshape(1, F), ln_g.reshape(1, F), ln_b.reshape(1, F),
      W_lin1T)


def _edge_stage_body(rbf_ref, d_ref, wrt_ref, b_ref, wm_ref):
    ea = jnp.dot(rbf_ref[...], wrt_ref[...],
                 preferred_element_type=jnp.float32) + b_ref[...]
    c = 0.5 * (jnp.cos(d_ref[0, 0] * (PI / CUTOFF)) + 1.0)
    w = lax.bitcast_convert_type((ea * c[:, None]).astype(jnp.bfloat16),
                                 jnp.uint16).astype(jnp.int32)
    wm_ref[0] = w[:, 0:FQ] | (w[:, FQ:2 * FQ] << 16)
    wm_ref[1] = w[:, 2 * FQ:3 * FQ] | (w[:, 3 * FQ:4 * FQ] << 16)


def _edge_stage(rbf, dists, W_r2mT, b_r2m):
    BE = 2000
    grid = E // BE
    return pl.pallas_call(
        _edge_stage_body,
        grid=(grid,),
        in_specs=[
            pl.BlockSpec((BE, R), lambda i: (i, 0)),
            pl.BlockSpec((1, 1, BE), lambda i: (i, 0, 0)),
            pl.BlockSpec((R, F), lambda i: (0, 0)),
            pl.BlockSpec((1, F), lambda i: (0, 0)),
        ],
        out_specs=pl.BlockSpec((NC, BE, FQ), lambda i: (0, i, 0)),
        out_shape=jax.ShapeDtypeStruct((NC, E, FQ), jnp.int32),
    )(rbf, dists.reshape(E // BE, 1, BE), W_r2mT, b_r2m.reshape(1, F))


def _out_stage_body(agg_ref, w2a_ref, w2b_ref, b_ref, wmt_ref, out_ref):
    t = jnp.dot(agg_ref[0], w2a_ref[...], preferred_element_type=jnp.float32)
    t = t + jnp.dot(agg_ref[1], w2b_ref[...], preferred_element_type=jnp.float32)
    t = t + b_ref[...]
    out_ref[...] = jnp.dot(t, wmt_ref[...], preferred_element_type=jnp.float32)


def _out_stage(agg, W2a, W2b, b_lin2, W_m2nT):
    BN = 2000
    grid = N // BN
    return pl.pallas_call(
        _out_stage_body,
        grid=(grid,),
        in_specs=[
            pl.BlockSpec((NC, BN, FH), lambda i: (0, i, 0)),
            pl.BlockSpec((FH, F), lambda i: (0, 0)),
            pl.BlockSpec((FH, F), lambda i: (0, 0)),
            pl.BlockSpec((1, F), lambda i: (0, 0)),
            pl.BlockSpec((F, H), lambda i: (0, 0)),
        ],
        out_specs=pl.BlockSpec((BN, H), lambda i: (i, 0)),
        out_shape=jax.ShapeDtypeStruct((N, H), jnp.float32),
    )(agg, W2a, W2b, b_lin2.reshape(1, F), W_m2nT)


def _sc_bin_body(src_ref, dst_ref,
                 sege_ref, segs_ref, segd_ref, cnt_ref,
                 sv, dv, cpad, sem, *stage):
    cid = lax.axis_index("c")
    sid = lax.axis_index("s")
    wid = cid * NS + sid
    cidN = cid * N
    cidE = cid * E
    iota = lax.iota(jnp.int32, L)

    ebase = sid * E_PER_TILE
    pltpu.sync_copy(src_ref.at[pl.ds(ebase, E_PER_TILE)], sv)
    pltpu.sync_copy(dst_ref.at[pl.ds(ebase, E_PER_TILE)], dv)
    eid0 = iota + (ebase + cidE)

    def _grp(g, cnts):
        dvg = dv[pl.ds(g * L, L)]
        svg = sv[pl.ds(g * L, L)]
        eid = eid0 + g * L
        sadj = svg + cidN
        outs = []
        for j in range(NB):
            dl = dvg - j * BKT
            m = (dl >= 0) & (dl < BKT)
            csum = plsc.cumsum(m.astype(jnp.int32))
            pos = jnp.minimum(cnts[j] + csum - 1, jnp.int32(CAP - 1))
            plsc.store_scatter(stage[3 * j], [pos], eid, mask=m)
            plsc.store_scatter(stage[3 * j + 1], [pos], sadj, mask=m)
            plsc.store_scatter(stage[3 * j + 2], [pos], dl, mask=m)
            outs.append(cnts[j] + csum[L - 1])
        return tuple(outs)

    z = jnp.int32(0)
    cnts = pl.loop(0, E_PER_TILE // L, init_carry=(z,) * NB)(_grp)

    cps = []
    cv0 = jnp.zeros((L,), jnp.int32)
    cv1 = jnp.zeros((L,), jnp.int32)
    for j in range(NB):
        cnt = jnp.minimum(cnts[j], jnp.int32(CAP))
        segrow = wid * NB + j
        cps.append(pltpu.async_copy(
            stage[3 * j], sege_ref.at[pl.ds(segrow * CAP, CAP)], sem))
        cps.append(pltpu.async_copy(
            stage[3 * j + 1], segs_ref.at[pl.ds(segrow * CAP, CAP)], sem))
        cps.append(pltpu.async_copy(
            stage[3 * j + 2], segd_ref.at[pl.ds(segrow * CAP, CAP)], sem))
        cv0 = cv0 + jnp.where(iota == j, cnt, 0)
        cv1 = cv1 + jnp.where(iota + L == j, cnt, 0)
    cpad[pl.ds(0, L)] = cv0
    cpad[pl.ds(L, L)] = cv1
    for cp in cps:
        cp.wait()
    pltpu.sync_copy(cpad, cnt_ref.at[pl.ds(wid * NB, NB)])


def _sc_acc_body(h1_ref, wm_ref, sege_ref, segs_ref, segd_ref, cnt_ref,
                 agg_out,
                 ste, sts, std, cbuf,
                 gath_a, wmg_a, gath_b, wmg_b, agg_v, sem, sem_a, sem_b):
    cid = lax.axis_index("c")
    sid = lax.axis_index("s")
    cidN = cid * N
    cidE = cid * E
    HI16 = jnp.int32(-65536)
    iota = lax.iota(jnp.int32, L)

    pltpu.sync_copy(cnt_ref.at[pl.ds(cid * NS * NB, NS * NB)], cbuf)

    def _bi(bi):
        b = sid + NS * bi
        blo = b * BKT

        @pl.loop(0, AGG_ROWS)
        def _z(r):
            for t in range(FH // L):
                agg_v[r, pl.ds(t * L, L)] = jnp.zeros((L,), jnp.float32)

        def _srctile(t):
            row0 = cbuf[pl.ds(t * NB, L)]
            row1 = cbuf[pl.ds(t * NB + L, L)]
            cnt = (jnp.sum(jnp.where(iota == b, row0, 0))
                   + jnp.sum(jnp.where(iota + L == b, row1, 0)))
            segrow = (cid * NS + t) * NB + b
            segbase = segrow * CAP
            c1 = pltpu.async_copy(sege_ref.at[pl.ds(segbase, CAP)], ste, sem)
            c2 = pltpu.async_copy(segs_ref.at[pl.ds(segbase, CAP)], sts, sem)
            c3 = pltpu.async_copy(segd_ref.at[pl.ds(segbase, CAP)], std, sem)
            c1.wait()
            c2.wait()
            c3.wait()
            nch = (cnt + (CH2 - 1)) // CH2

            @pl.loop(0, CAP // L)
            def _msk(q):
                pos = q * L + iota
                valid = pos < cnt
                sts[pl.ds(q * L, L)] = jnp.where(
                    valid, sts[pl.ds(q * L, L)], cidN)
                std[pl.ds(q * L, L)] = jnp.where(
                    valid, std[pl.ds(q * L, L)], jnp.int32(BKT))
                ste[pl.ds(q * L, L)] = jnp.where(
                    valid, ste[pl.ds(q * L, L)], cidE)

            def _issue(g, gbuf, wbuf, semx):
                off = g * CH2
                pltpu.async_copy(
                    h1_ref.at[plsc.Indices(sts.at[pl.ds(off, CH2)])],
                    gbuf, semx)
                pltpu.async_copy(
                    wm_ref.at[plsc.Indices(ste.at[pl.ds(off, CH2)])],
                    wbuf, semx)

            def _wait(gbuf, wbuf, semx):
                pltpu.make_async_copy(
                    h1_ref.at[pl.ds(0, CH2)], gbuf, semx).wait()
                pltpu.make_async_copy(
                    wm_ref.at[pl.ds(0, CH2)], wbuf, semx).wait()

            def _acc(g, gbuf, wbuf):
                off = g * CH2

                @pl.loop(0, CH2 // L)
                def _q(q):
                    dvec = std[pl.ds(off + q * L, L)]
                    for k in range(L):
                        e = q * L + k
                        r = dvec[k]
                        for w in range(FQ // L):
                            hw = gbuf[e, pl.ds(w * L, L)]
                            ww = wbuf[e, pl.ds(w * L, L)]
                            h_lo = plsc.bitcast(hw << 16, jnp.float32)
                            h_hi = plsc.bitcast(hw & HI16, jnp.float32)
                            w_lo = plsc.bitcast(ww << 16, jnp.float32)
                            w_hi = plsc.bitcast(ww & HI16, jnp.float32)
                            plsc.addupdate(
                                agg_v.at[r, pl.ds(w * L, L)], h_lo * w_lo)
                            plsc.addupdate(
                                agg_v.at[r, pl.ds(FQ + w * L, L)], h_hi * w_hi)

            @pl.when(nch > 0)
            def _pro():
                _issue(0, gath_a, wmg_a, sem_a)

            def _half(h):
                ga = 2 * h
                gb = 2 * h + 1

                @pl.when(gb < nch)
                def _ib():
                    _issue(gb, gath_b, wmg_b, sem_b)
                _wait(gath_a, wmg_a, sem_a)
                _acc(ga, gath_a, wmg_a)

                @pl.when(gb < nch)
                def _db():
                    @pl.when(gb + 1 < nch)
                    def _ia():
                        _issue(gb + 1, gath_a, wmg_a, sem_a)
                    _wait(gath_b, wmg_b, sem_b)
                    _acc(gb, gath_b, wmg_b)

            pl.loop(0, (nch + 1) // 2)(_half)

        pl.loop(0, NS)(_srctile)
        pltpu.sync_copy(agg_v.at[pl.ds(0, BKT)],
                        agg_out.at[cid, pl.ds(blo, BKT)])

    pl.loop(0, 2)(_bi)


def _sc_stage(h1p, wmp, src, dst):
    mesh = plsc.VectorSubcoreMesh(core_axis_name="c", subcore_axis_name="s")
    sege, segs, segd, cnts = pl.kernel(
        _sc_bin_body,
        out_type=(
            jax.ShapeDtypeStruct((NC * NS * NB * CAP,), jnp.int32),
            jax.ShapeDtypeStruct((NC * NS * NB * CAP,), jnp.int32),
            jax.ShapeDtypeStruct((NC * NS * NB * CAP,), jnp.int32),
            jax.ShapeDtypeStruct((NC * NS * NB,), jnp.int32),
        ),
        mesh=mesh,
        compiler_params=pltpu.CompilerParams(needs_layout_passes=False),
        scratch_types=[
            pltpu.VMEM((E_PER_TILE,), jnp.int32),
            pltpu.VMEM((E_PER_TILE,), jnp.int32),
            pltpu.VMEM((NB,), jnp.int32),
            pltpu.SemaphoreType.DMA,
        ] + [pltpu.VMEM((CAP,), jnp.int32)] * (3 * NB),
    )(src, dst)
    return pl.kernel(
        _sc_acc_body,
        out_type=jax.ShapeDtypeStruct((NC, NPAD, FH), jnp.float32),
        mesh=mesh,
        compiler_params=pltpu.CompilerParams(needs_layout_passes=False),
        scratch_types=[
            pltpu.VMEM((CAP,), jnp.int32),
            pltpu.VMEM((CAP,), jnp.int32),
            pltpu.VMEM((CAP,), jnp.int32),
            pltpu.VMEM((NS * NB,), jnp.int32),
            pltpu.VMEM((CH2, FQ), jnp.int32),
            pltpu.VMEM((CH2, FQ), jnp.int32),
            pltpu.VMEM((CH2, FQ), jnp.int32),
            pltpu.VMEM((CH2, FQ), jnp.int32),
            pltpu.VMEM((AGG_ROWS, FH), jnp.float32),
            pltpu.SemaphoreType.DMA,
            pltpu.SemaphoreType.DMA,
            pltpu.SemaphoreType.DMA,
        ],
    )(h1p, wmp, sege, segs, segd, cnts)


@jax.jit
def kernel(x, rbf, dists, edge_index, W_n2m, b_n2m, ln_g, ln_b, W_r2m, b_r2m,
           W_lin1, W_lin2, b_lin2, W_m2n):
    src = edge_index[0].astype(jnp.int32)
    dst = edge_index[1].astype(jnp.int32)

    h1p = _node_stage(x, W_n2m.T, b_n2m, ln_g, ln_b, W_lin1.T)
    wmp = _edge_stage(rbf, dists, W_r2m.T, b_r2m)

    agg = _sc_stage(h1p.reshape(NC * N, FQ), wmp.reshape(NC * E, FQ), src, dst)

    W2a = W_lin2[:, :FH].T
    W2b = W_lin2[:, FH:].T
    return _out_stage(agg, W2a, W2b, b_lin2, W_m2n.T)


# speedup vs baseline: 1.0730x; 1.0730x over previous
import functools

import numpy as np
import jax
import jax.numpy as jnp
from jax import lax
from jax.experimental import pallas as pl
from jax.experimental.pallas import tpu as pltpu
from jax.experimental.pallas import tpu_sc as plsc

PI = np.pi
CUTOFF = 5.0

N, E, H, F, R = 10000, 160000, 512, 512, 64
NC, NS, L = 2, 16, 16
FH = F // NC
FQ = F // 4
NB = 2 * NS
BKT = 320
NPAD = NB * BKT
E_PER_TILE = E // NS
CAP = 560
CH2 = 32
AGG_ROWS = BKT + 1


def _node_stage_body(x_ref, wnt_ref, b_ref, g_ref, beta_ref, wl1t_ref, h1_ref):
    xb = x_ref[...]
    h = jnp.dot(xb, wnt_ref[...], preferred_element_type=jnp.float32) + b_ref[...]
    mu = jnp.mean(h, axis=-1, keepdims=True)
    var = jnp.mean((h - mu) ** 2, axis=-1, keepdims=True)
    hn = (h - mu) * lax.rsqrt(var + 1e-5) * g_ref[...] + beta_ref[...]
    h1 = jnp.dot(hn, wl1t_ref[...], preferred_element_type=jnp.float32)
    w = lax.bitcast_convert_type(h1.astype(jnp.bfloat16),
                                 jnp.uint16).astype(jnp.int32)
    h1_ref[0] = w[:, 0:FQ] | (w[:, FQ:2 * FQ] << 16)
    h1_ref[1] = w[:, 2 * FQ:3 * FQ] | (w[:, 3 * FQ:4 * FQ] << 16)


def _node_stage(x, W_n2mT, b_n2m, ln_g, ln_b, W_lin1T):
    BN = 2000
    grid = N // BN
    return pl.pallas_call(
        _node_stage_body,
        grid=(grid,),
        in_specs=[
            pl.BlockSpec((BN, H), lambda i: (i, 0)),
            pl.BlockSpec((H, F), lambda i: (0, 0)),
            pl.BlockSpec((1, F), lambda i: (0, 0)),
            pl.BlockSpec((1, F), lambda i: (0, 0)),
            pl.BlockSpec((1, F), lambda i: (0, 0)),
            pl.BlockSpec((F, F), lambda i: (0, 0)),
        ],
        out_specs=pl.BlockSpec((NC, BN, FQ), lambda i: (0, i, 0)),
        out_shape=jax.ShapeDtypeStruct((NC, N, FQ), jnp.int32),
    )(x, W_n2mT, b_n2m.reshape(1, F), ln_g.reshape(1, F), ln_b.reshape(1, F),
      W_lin1T)


def _edge_stage_body(rbf_ref, d_ref, wrt_ref, b_ref, wm_ref):
    ea = jnp.dot(rbf_ref[...], wrt_ref[...],
                 preferred_element_type=jnp.float32) + b_ref[...]
    c = 0.5 * (jnp.cos(d_ref[0, 0] * (PI / CUTOFF)) + 1.0)
    w = lax.bitcast_convert_type((ea * c[:, None]).astype(jnp.bfloat16),
                                 jnp.uint16).astype(jnp.int32)
    wm_ref[0] = w[:, 0:FQ] | (w[:, FQ:2 * FQ] << 16)
    wm_ref[1] = w[:, 2 * FQ:3 * FQ] | (w[:, 3 * FQ:4 * FQ] << 16)


def _edge_stage(rbf, dists, W_r2mT, b_r2m):
    BE = 2000
    grid = E // BE
    return pl.pallas_call(
        _edge_stage_body,
        grid=(grid,),
        in_specs=[
            pl.BlockSpec((BE, R), lambda i: (i, 0)),
            pl.BlockSpec((1, 1, BE), lambda i: (i, 0, 0)),
            pl.BlockSpec((R, F), lambda i: (0, 0)),
            pl.BlockSpec((1, F), lambda i: (0, 0)),
        ],
        out_specs=pl.BlockSpec((NC, BE, FQ), lambda i: (0, i, 0)),
        out_shape=jax.ShapeDtypeStruct((NC, E, FQ), jnp.int32),
    )(rbf, dists.reshape(E // BE, 1, BE), W_r2mT, b_r2m.reshape(1, F))


def _out_stage_body(agg_ref, w2a_ref, w2b_ref, b_ref, wmt_ref, out_ref):
    t = jnp.dot(agg_ref[0], w2a_ref[...], preferred_element_type=jnp.float32)
    t = t + jnp.dot(agg_ref[1], w2b_ref[...], preferred_element_type=jnp.float32)
    t = t + b_ref[...]
    out_ref[...] = jnp.dot(t, wmt_ref[...], preferred_element_type=jnp.float32)


def _out_stage(agg, W2a, W2b, b_lin2, W_m2nT):
    BN = 2000
    grid = N // BN
    return pl.pallas_call(
        _out_stage_body,
        grid=(grid,),
        in_specs=[
            pl.BlockSpec((NC, BN, FH), lambda i: (0, i, 0)),
            pl.BlockSpec((FH, F), lambda i: (0, 0)),
            pl.BlockSpec((FH, F), lambda i: (0, 0)),
            pl.BlockSpec((1, F), lambda i: (0, 0)),
            pl.BlockSpec((F, H), lambda i: (0, 0)),
        ],
        out_specs=pl.BlockSpec((BN, H), lambda i: (i, 0)),
        out_shape=jax.ShapeDtypeStruct((N, H), jnp.float32),
    )(agg, W2a, W2b, b_lin2.reshape(1, F), W_m2nT)


def _sc_bin_body(src_ref, dst_ref,
                 sege_ref, segs_ref, segd_ref, cnt_ref,
                 sv, dv, cpad, sem, *stage):
    cid = lax.axis_index("c")
    sid = lax.axis_index("s")
    wid = cid * NS + sid
    cidN = cid * N
    cidE = cid * E
    iota = lax.iota(jnp.int32, L)

    ebase = sid * E_PER_TILE
    pltpu.sync_copy(src_ref.at[pl.ds(ebase, E_PER_TILE)], sv)
    pltpu.sync_copy(dst_ref.at[pl.ds(ebase, E_PER_TILE)], dv)
    eid0 = iota + (ebase + cidE)

    def _grp(g, cnts):
        dvg = dv[pl.ds(g * L, L)]
        svg = sv[pl.ds(g * L, L)]
        eid = eid0 + g * L
        sadj = svg + cidN
        outs = []
        for j in range(NB):
            dl = dvg - j * BKT
            m = (dl >= 0) & (dl < BKT)
            csum = plsc.cumsum(m.astype(jnp.int32))
            pos = jnp.minimum(cnts[j] + csum - 1, jnp.int32(CAP - 1))
            plsc.store_scatter(stage[3 * j], [pos], eid, mask=m)
            plsc.store_scatter(stage[3 * j + 1], [pos], sadj, mask=m)
            plsc.store_scatter(stage[3 * j + 2], [pos], dl, mask=m)
            outs.append(cnts[j] + csum[L - 1])
        return tuple(outs)

    z = jnp.int32(0)
    cnts = pl.loop(0, E_PER_TILE // L, init_carry=(z,) * NB)(_grp)

    cps = []
    cv0 = jnp.zeros((L,), jnp.int32)
    cv1 = jnp.zeros((L,), jnp.int32)
    for j in range(NB):
        cnt = jnp.minimum(cnts[j], jnp.int32(CAP))
        segrow = (cid * NB + j) * NS + sid
        cps.append(pltpu.async_copy(
            stage[3 * j], sege_ref.at[pl.ds(segrow * CAP, CAP)], sem))
        cps.append(pltpu.async_copy(
            stage[3 * j + 1], segs_ref.at[pl.ds(segrow * CAP, CAP)], sem))
        cps.append(pltpu.async_copy(
            stage[3 * j + 2], segd_ref.at[pl.ds(segrow * CAP, CAP)], sem))
        cv0 = cv0 + jnp.where(iota == j, cnt, 0)
        cv1 = cv1 + jnp.where(iota + L == j, cnt, 0)
    cpad[pl.ds(0, L)] = cv0
    cpad[pl.ds(L, L)] = cv1
    for cp in cps:
        cp.wait()
    pltpu.sync_copy(cpad, cnt_ref.at[pl.ds(wid * NB, NB)])


def _sc_acc_body(h1_ref, wm_ref, sege_ref, segs_ref, segd_ref, cnt_ref,
                 agg_out,
                 ste, sts, std, cbuf,
                 gath_a, wmg_a, gath_b, wmg_b, agg_v, sem, sem_a, sem_b):
    cid = lax.axis_index("c")
    sid = lax.axis_index("s")
    cidN = cid * N
    cidE = cid * E
    HI16 = jnp.int32(-65536)
    iota = lax.iota(jnp.int32, L)

    pltpu.sync_copy(cnt_ref.at[pl.ds(cid * NS * NB, NS * NB)], cbuf)

    def _bi(bi):
        b = sid + NS * bi
        blo = b * BKT

        bktbase = (cid * NB + b) * NS * CAP
        c1 = pltpu.async_copy(sege_ref.at[pl.ds(bktbase, NS * CAP)], ste, sem)
        c2 = pltpu.async_copy(segs_ref.at[pl.ds(bktbase, NS * CAP)], sts, sem)
        c3 = pltpu.async_copy(segd_ref.at[pl.ds(bktbase, NS * CAP)], std, sem)

        @pl.loop(0, AGG_ROWS)
        def _z(r):
            for t in range(FH // L):
                agg_v[r, pl.ds(t * L, L)] = jnp.zeros((L,), jnp.float32)

        c1.wait()
        c2.wait()
        c3.wait()

        def _srctile(t):
            row0 = cbuf[pl.ds(t * NB, L)]
            row1 = cbuf[pl.ds(t * NB + L, L)]
            cnt = (jnp.sum(jnp.where(iota == b, row0, 0))
                   + jnp.sum(jnp.where(iota + L == b, row1, 0)))
            tbase = t * CAP
            nch = (cnt + (CH2 - 1)) // CH2

            @pl.loop(0, CAP // L)
            def _msk(q):
                pos = q * L + iota
                valid = pos < cnt
                sts[pl.ds(tbase + q * L, L)] = jnp.where(
                    valid, sts[pl.ds(tbase + q * L, L)], cidN)
                std[pl.ds(tbase + q * L, L)] = jnp.where(
                    valid, std[pl.ds(tbase + q * L, L)], jnp.int32(BKT))
                ste[pl.ds(tbase + q * L, L)] = jnp.where(
                    valid, ste[pl.ds(tbase + q * L, L)], cidE)

            def _issue(g, gbuf, wbuf, semx):
                off = tbase + g * CH2
                pltpu.async_copy(
                    h1_ref.at[plsc.Indices(sts.at[pl.ds(off, CH2)])],
                    gbuf, semx)
                pltpu.async_copy(
                    wm_ref.at[plsc.Indices(ste.at[pl.ds(off, CH2)])],
                    wbuf, semx)

            def _wait(gbuf, wbuf, semx):
                pltpu.make_async_copy(
                    h1_ref.at[pl.ds(0, CH2)], gbuf, semx).wait()
                pltpu.make_async_copy(
                    wm_ref.at[pl.ds(0, CH2)], wbuf, semx).wait()

            def _acc(g, gbuf, wbuf):
                off = tbase + g * CH2

                @pl.loop(0, CH2 // L)
                def _q(q):
                    dvec = std[pl.ds(off + q * L, L)]
                    for k in range(L):
                        e = q * L + k
                        r = dvec[k]
                        for w in range(FQ // L):
                            hw = gbuf[e, pl.ds(w * L, L)]
                            ww = wbuf[e, pl.ds(w * L, L)]
                            h_lo = plsc.bitcast(hw << 16, jnp.float32)
                            h_hi = plsc.bitcast(hw & HI16, jnp.float32)
                            w_lo = plsc.bitcast(ww << 16, jnp.float32)
                            w_hi = plsc.bitcast(ww & HI16, jnp.float32)
                            plsc.addupdate(
                                agg_v.at[r, pl.ds(w * L, L)], h_lo * w_lo)
                            plsc.addupdate(
                                agg_v.at[r, pl.ds(FQ + w * L, L)], h_hi * w_hi)

            @pl.when(nch > 0)
            def _pro():
                _issue(0, gath_a, wmg_a, sem_a)

            def _half(h):
                ga = 2 * h
                gb = 2 * h + 1

                @pl.when(gb < nch)
                def _ib():
                    _issue(gb, gath_b, wmg_b, sem_b)
                _wait(gath_a, wmg_a, sem_a)
                _acc(ga, gath_a, wmg_a)

                @pl.when(gb < nch)
                def _db():
                    @pl.when(gb + 1 < nch)
                    def _ia():
                        _issue(gb + 1, gath_a, wmg_a, sem_a)
                    _wait(gath_b, wmg_b, sem_b)
                    _acc(gb, gath_b, wmg_b)

            pl.loop(0, (nch + 1) // 2)(_half)

        pl.loop(0, NS)(_srctile)
        pltpu.sync_copy(agg_v.at[pl.ds(0, BKT)],
                        agg_out.at[cid, pl.ds(blo, BKT)])

    pl.loop(0, 2)(_bi)


def _sc_stage(h1p, wmp, src, dst):
    mesh = plsc.VectorSubcoreMesh(core_axis_name="c", subcore_axis_name="s")
    sege, segs, segd, cnts = pl.kernel(
        _sc_bin_body,
        out_type=(
            jax.ShapeDtypeStruct((NC * NS * NB * CAP,), jnp.int32),
            jax.ShapeDtypeStruct((NC * NS * NB * CAP,), jnp.int32),
            jax.ShapeDtypeStruct((NC * NS * NB * CAP,), jnp.int32),
            jax.ShapeDtypeStruct((NC * NS * NB,), jnp.int32),
        ),
        mesh=mesh,
        compiler_params=pltpu.CompilerParams(needs_layout_passes=False),
        scratch_types=[
            pltpu.VMEM((E_PER_TILE,), jnp.int32),
            pltpu.VMEM((E_PER_TILE,), jnp.int32),
            pltpu.VMEM((NB,), jnp.int32),
            pltpu.SemaphoreType.DMA,
        ] + [pltpu.VMEM((CAP,), jnp.int32)] * (3 * NB),
    )(src, dst)
    return pl.kernel(
        _sc_acc_body,
        out_type=jax.ShapeDtypeStruct((NC, NPAD, FH), jnp.float32),
        mesh=mesh,
        compiler_params=pltpu.CompilerParams(needs_layout_passes=False),
        scratch_types=[
            pltpu.VMEM((NS * CAP,), jnp.int32),
            pltpu.VMEM((NS * CAP,), jnp.int32),
            pltpu.VMEM((NS * CAP,), jnp.int32),
            pltpu.VMEM((NS * NB,), jnp.int32),
            pltpu.VMEM((CH2, FQ), jnp.int32),
            pltpu.VMEM((CH2, FQ), jnp.int32),
            pltpu.VMEM((CH2, FQ), jnp.int32),
            pltpu.VMEM((CH2, FQ), jnp.int32),
            pltpu.VMEM((AGG_ROWS, FH), jnp.float32),
            pltpu.SemaphoreType.DMA,
            pltpu.SemaphoreType.DMA,
            pltpu.SemaphoreType.DMA,
        ],
    )(h1p, wmp, sege, segs, segd, cnts)


@jax.jit
def kernel(x, rbf, dists, edge_index, W_n2m, b_n2m, ln_g, ln_b, W_r2m, b_r2m,
           W_lin1, W_lin2, b_lin2, W_m2n):
    src = edge_index[0].astype(jnp.int32)
    dst = edge_index[1].astype(jnp.int32)

    h1p = _node_stage(x, W_n2m.T, b_n2m, ln_g, ln_b, W_lin1.T)
    wmp = _edge_stage(rbf, dists, W_r2m.T, b_r2m)

    agg = _sc_stage(h1p.reshape(NC * N, FQ), wmp.reshape(NC * E, FQ), src, dst)

    W2a = W_lin2[:, :FH].T
    W2b = W_lin2[:, FH:].T
    return _out_stage(agg, W2a, W2b, b_lin2, W_m2n.T)
